# Initial kernel scaffold; baseline (speedup 1.0000x reference)
import jax
import jax.numpy as jnp
from jax.experimental import pallas as pl


def kernel(chi, hardness, segment_ids, total_charge):
    raise NotImplementedError("write your pallas kernel here")


# speedup vs baseline: 150.9519x; 150.9519x over previous
import functools

import jax
import jax.numpy as jnp
from jax import lax
from jax.experimental import pallas as pl
from jax.experimental.pallas import tpu as pltpu
from jax.experimental.pallas import tpu_sc as plsc

N = 1_600_000
B = 50_000
NC, NS, L = 2, 16, 16
NW = NC * NS
CHUNK = N // NW
T = 2_000
NT = CHUNK // T
BP = 51_200
SB = BP // NS

_mesh = plsc.VectorSubcoreMesh(core_axis_name="c", subcore_axis_name="s")


def _zero_fill(buf, n):
    zeros = jnp.zeros((L,), jnp.float32)

    def body(i, _):
        buf[pl.ds(i * L, L)] = zeros
        return 0

    lax.fori_loop(0, n // L, body, 0)


def _partials_body(chi_hbm, hard_hbm, ids_hbm, out_hbm,
                   idx_v, a_v, b_v, hinv_v, chinv_v, z_v, acc_h, acc_c):
    cid = lax.axis_index("c")
    sid = lax.axis_index("s")
    wid = sid * NC + cid

    _zero_fill(z_v, SB)
    pltpu.sync_copy(z_v, acc_h.at[pl.ds(sid * SB, SB)])
    pltpu.sync_copy(z_v, acc_c.at[pl.ds(sid * SB, SB)])
    plsc.subcore_barrier()

    def tile(t, _):
        base = wid * CHUNK + t * T
        pltpu.sync_copy(ids_hbm.at[pl.ds(base, T)], idx_v)
        pltpu.sync_copy(chi_hbm.at[pl.ds(base, T)], a_v)
        pltpu.sync_copy(hard_hbm.at[pl.ds(base, T)], b_v)

        def vec(i, _):
            sl = pl.ds(i * L, L)
            hinv = 1.0 / b_v[sl]
            hinv_v[sl] = hinv
            chinv_v[sl] = a_v[sl] * hinv
            return 0

        lax.fori_loop(0, T // L, vec, 0)
        pltpu.sync_copy(hinv_v, acc_h.at[idx_v], add=True)
        pltpu.sync_copy(chinv_v, acc_c.at[idx_v], add=True)
        return 0

    lax.fori_loop(0, NT, tile, 0)
    plsc.subcore_barrier()

    sl = pl.ds(sid * SB, SB)
    pltpu.sync_copy(acc_h.at[sl], out_hbm.at[cid, 0, sl])
    pltpu.sync_copy(acc_c.at[sl], out_hbm.at[cid, 1, sl])


_partials = functools.partial(
    pl.kernel,
    out_type=jax.ShapeDtypeStruct((NC, 2, BP), jnp.float32),
    mesh=_mesh,
    scratch_types=[
        pltpu.VMEM((T,), jnp.int32),
        pltpu.VMEM((T,), jnp.float32),
        pltpu.VMEM((T,), jnp.float32),
        pltpu.VMEM((T,), jnp.float32),
        pltpu.VMEM((T,), jnp.float32),
        pltpu.VMEM((SB,), jnp.float32),
        pltpu.VMEM_SHARED((BP,), jnp.float32),
        pltpu.VMEM_SHARED((BP,), jnp.float32),
    ],
)(_partials_body)


def _charges_body(chi_hbm, hard_hbm, ids_hbm, tc_hbm, parts_hbm, out_hbm,
                  idx_v, a_v, b_v, g_v, o_v, pa_v, pb_v, pc_v, pd_v, tc_v,
                  r_sh):
    cid = lax.axis_index("c")
    sid = lax.axis_index("s")
    wid = sid * NC + cid

    sl = pl.ds(sid * SB, SB)
    pltpu.sync_copy(parts_hbm.at[0, 0, sl], pa_v)
    pltpu.sync_copy(parts_hbm.at[1, 0, sl], pb_v)
    pltpu.sync_copy(parts_hbm.at[0, 1, sl], pc_v)
    pltpu.sync_copy(parts_hbm.at[1, 1, sl], pd_v)
    pltpu.sync_copy(tc_hbm, tc_v)
    tc = tc_v[...]

    def rvec(i, _):
        s = pl.ds(i * L, L)
        sh = pa_v[s] + pb_v[s]
        sc = pc_v[s] + pd_v[s]
        pa_v[s] = (tc + sc) / sh
        return 0

    lax.fori_loop(0, SB // L, rvec, 0)
    pltpu.sync_copy(pa_v, r_sh.at[sl])
    plsc.subcore_barrier()

    def tile(t, _):
        base = wid * CHUNK + t * T
        pltpu.sync_copy(ids_hbm.at[pl.ds(base, T)], idx_v)
        pltpu.sync_copy(chi_hbm.at[pl.ds(base, T)], a_v)
        pltpu.sync_copy(hard_hbm.at[pl.ds(base, T)], b_v)
        pltpu.sync_copy(r_sh.at[idx_v], g_v)

        def vec(i, _):
            s = pl.ds(i * L, L)
            o_v[s] = (g_v[s] - a_v[s]) / b_v[s]
            return 0

        lax.fori_loop(0, T // L, vec, 0)
        pltpu.sync_copy(o_v, out_hbm.at[pl.ds(base, T)])
        return 0

    lax.fori_loop(0, NT, tile, 0)


_charges = functools.partial(
    pl.kernel,
    out_type=jax.ShapeDtypeStruct((N,), jnp.float32),
    mesh=_mesh,
    scratch_types=[
        pltpu.VMEM((T,), jnp.int32),
        pltpu.VMEM((T,), jnp.float32),
        pltpu.VMEM((T,), jnp.float32),
        pltpu.VMEM((T,), jnp.float32),
        pltpu.VMEM((T,), jnp.float32),
        pltpu.VMEM((SB,), jnp.float32),
        pltpu.VMEM((SB,), jnp.float32),
        pltpu.VMEM((SB,), jnp.float32),
        pltpu.VMEM((SB,), jnp.float32),
        pltpu.VMEM((L,), jnp.float32),
        pltpu.VMEM_SHARED((BP,), jnp.float32),
    ],
)(_charges_body)


def kernel(chi, hardness, segment_ids, total_charge):
    tc_vec = jnp.broadcast_to(total_charge, (L,)).astype(jnp.float32)
    parts = _partials(chi, hardness, segment_ids)
    return _charges(chi, hardness, segment_ids, tc_vec, parts)


# speedup vs baseline: 292.6578x; 1.9387x over previous
import functools

import jax
import jax.numpy as jnp
from jax import lax
from jax.experimental import pallas as pl
from jax.experimental.pallas import tpu as pltpu
from jax.experimental.pallas import tpu_sc as plsc

N = 1_600_000
B = 50_000
NC, NS, L = 2, 16, 16
NW = NC * NS
CHUNK = N // NW
T = 2_000
NT = CHUNK // T
NBUF = 4
BP = 51_200
SB = BP // NS

_mesh = plsc.VectorSubcoreMesh(core_axis_name="c", subcore_axis_name="s")


def _zero_fill(buf, n):
    zeros = jnp.zeros((L,), jnp.float32)

    @plsc.parallel_loop(0, n // L, unroll=5)
    def _(i):
        buf[pl.ds(i * L, L)] = zeros


def _partials_body(chi_hbm, hard_hbm, ids_hbm, out_hbm,
                   idx_v, a_v, b_v, hv_v, cv_v, z_v, acc_h, acc_c,
                   lsems, ssems):
    cid = lax.axis_index("c")
    sid = lax.axis_index("s")
    wid = sid * NC + cid

    _zero_fill(z_v, SB)
    pltpu.sync_copy(z_v, acc_h.at[pl.ds(sid * SB, SB)])
    pltpu.sync_copy(z_v, acc_c.at[pl.ds(sid * SB, SB)])
    plsc.subcore_barrier()

    def issue_loads(t):
        s = t % NBUF
        sl = pl.ds(wid * CHUNK + t * T, T)
        return (pltpu.async_copy(ids_hbm.at[sl], idx_v[s], lsems[s]),
                pltpu.async_copy(chi_hbm.at[sl], a_v[s], lsems[s]),
                pltpu.async_copy(hard_hbm.at[sl], b_v[s], lsems[s]))

    loads = {0: issue_loads(0), 1: issue_loads(1)}
    scats = {}
    for t in range(NT):
        s = t % NBUF
        if t >= 2:
            for d in scats.pop(t - 2):
                d.wait()
        if t + 2 < NT:
            loads[t + 2] = issue_loads(t + 2)
        for d in loads.pop(t):
            d.wait()

        ar, br, hr, cr = a_v[s], b_v[s], hv_v[s], cv_v[s]

        @plsc.parallel_loop(0, T // L, unroll=5)
        def _(i):
            sl = pl.ds(i * L, L)
            hinv = 1.0 / br[sl]
            hr[sl] = hinv
            cr[sl] = ar[sl] * hinv

        ir = idx_v[s]
        scats[t] = (
            pltpu.async_copy(hr, acc_h.at[ir], ssems[s], add=True),
            pltpu.async_copy(cr, acc_c.at[ir], ssems[s], add=True))
    for t in sorted(scats):
        for d in scats[t]:
            d.wait()

    plsc.subcore_barrier()
    sl = pl.ds(sid * SB, SB)
    pltpu.sync_copy(acc_h.at[sl], out_hbm.at[cid, 0, sl])
    pltpu.sync_copy(acc_c.at[sl], out_hbm.at[cid, 1, sl])


_partials = functools.partial(
    pl.kernel,
    out_type=jax.ShapeDtypeStruct((NC, 2, BP), jnp.float32),
    mesh=_mesh,
    scratch_types=[
        [pltpu.VMEM((T,), jnp.int32)] * NBUF,
        [pltpu.VMEM((T,), jnp.float32)] * NBUF,
        [pltpu.VMEM((T,), jnp.float32)] * NBUF,
        [pltpu.VMEM((T,), jnp.float32)] * NBUF,
        [pltpu.VMEM((T,), jnp.float32)] * NBUF,
        pltpu.VMEM((SB,), jnp.float32),
        pltpu.VMEM_SHARED((BP,), jnp.float32),
        pltpu.VMEM_SHARED((BP,), jnp.float32),
        [pltpu.SemaphoreType.DMA] * NBUF,
        [pltpu.SemaphoreType.DMA] * NBUF,
    ],
)(_partials_body)


def _charges_body(chi_hbm, hard_hbm, ids_hbm, tc_hbm, parts_hbm, out_hbm,
                  idx_v, a_v, b_v, g_v, o_v, pa_v, pb_v, pc_v, pd_v, tc_v,
                  r_sh, lsems, gsems, stsems):
    cid = lax.axis_index("c")
    sid = lax.axis_index("s")
    wid = sid * NC + cid

    sl = pl.ds(sid * SB, SB)
    pltpu.sync_copy(parts_hbm.at[0, 0, sl], pa_v)
    pltpu.sync_copy(parts_hbm.at[1, 0, sl], pb_v)
    pltpu.sync_copy(parts_hbm.at[0, 1, sl], pc_v)
    pltpu.sync_copy(parts_hbm.at[1, 1, sl], pd_v)
    pltpu.sync_copy(tc_hbm, tc_v)
    tc = tc_v[...]

    @plsc.parallel_loop(0, SB // L, unroll=5)
    def _(i):
        s = pl.ds(i * L, L)
        sh = pa_v[s] + pb_v[s]
        sc = pc_v[s] + pd_v[s]
        pa_v[s] = (tc + sc) / sh

    pltpu.sync_copy(pa_v, r_sh.at[sl])
    plsc.subcore_barrier()

    def issue_loads(t):
        s = t % NBUF
        sl = pl.ds(wid * CHUNK + t * T, T)
        return (pltpu.async_copy(ids_hbm.at[sl], idx_v[s], lsems[s]),
                pltpu.async_copy(chi_hbm.at[sl], a_v[s], lsems[s]),
                pltpu.async_copy(hard_hbm.at[sl], b_v[s], lsems[s]))

    loads = {0: issue_loads(0), 1: issue_loads(1)}
    gathers, stores = {}, {}
    for t in range(NT + 1):
        if t < NT:
            s = t % NBUF
            if t >= 2:
                stores.pop(t - 2).wait()
            if t + 2 < NT:
                loads[t + 2] = issue_loads(t + 2)
            for d in loads.pop(t):
                d.wait()
            gathers[t] = pltpu.async_copy(
                r_sh.at[idx_v[s]], g_v[s], gsems[s])
        if t >= 1:
            u = t - 1
            su = u % NBUF
            gathers.pop(u).wait()
            ar, br, gr, orr = a_v[su], b_v[su], g_v[su], o_v[su]

            @plsc.parallel_loop(0, T // L, unroll=5)
            def _(i):
                s2 = pl.ds(i * L, L)
                orr[s2] = (gr[s2] - ar[s2]) / br[s2]

            stores[u] = pltpu.async_copy(
                orr, out_hbm.at[pl.ds(wid * CHUNK + u * T, T)], stsems[su])
    for t in sorted(stores):
        stores[t].wait()


_charges = functools.partial(
    pl.kernel,
    out_type=jax.ShapeDtypeStruct((N,), jnp.float32),
    mesh=_mesh,
    scratch_types=[
        [pltpu.VMEM((T,), jnp.int32)] * NBUF,
        [pltpu.VMEM((T,), jnp.float32)] * NBUF,
        [pltpu.VMEM((T,), jnp.float32)] * NBUF,
        [pltpu.VMEM((T,), jnp.float32)] * NBUF,
        [pltpu.VMEM((T,), jnp.float32)] * NBUF,
        pltpu.VMEM((SB,), jnp.float32),
        pltpu.VMEM((SB,), jnp.float32),
        pltpu.VMEM((SB,), jnp.float32),
        pltpu.VMEM((SB,), jnp.float32),
        pltpu.VMEM((L,), jnp.float32),
        pltpu.VMEM_SHARED((BP,), jnp.float32),
        [pltpu.SemaphoreType.DMA] * NBUF,
        [pltpu.SemaphoreType.DMA] * NBUF,
        [pltpu.SemaphoreType.DMA] * NBUF,
    ],
)(_charges_body)


def kernel(chi, hardness, segment_ids, total_charge):
    tc_vec = jnp.broadcast_to(total_charge, (L,)).astype(jnp.float32)
    parts = _partials(chi, hardness, segment_ids)
    return _charges(chi, hardness, segment_ids, tc_vec, parts)
